# Initial kernel scaffold; baseline (speedup 1.0000x reference)
import jax
import jax.numpy as jnp
from jax.experimental import pallas as pl


def kernel(x, edge_index, W1, b1, Ws1, W2, b2, W3, b3, Ws2, W4, b4, Wf1, bf1, Wf2, bf2):
    raise NotImplementedError("write your pallas kernel here")


# speedup vs baseline: 33.1689x; 33.1689x over previous
import functools

import jax
import jax.numpy as jnp
from jax import lax
from jax.experimental import pallas as pl
from jax.experimental.pallas import tpu as pltpu
from jax.experimental.pallas import tpu_sc as plsc

N = 32832
E = N * 32
IB = 128
NC, NS = 2, 16
NW = NC * NS
NB16 = 16
NB32 = 8
NCH16 = E // (NB16 * IB)
NCH32 = E // (NB32 * IB)
ZR = 2048
TAIL = N - (N // ZR) * ZR

_MESH = plsc.VectorSubcoreMesh(core_axis_name="c", subcore_axis_name="s")
_SC_PARAMS = pltpu.CompilerParams(use_tc_tiling_on_sc=False)


def _zero_acc(zeros_hbm, acc, sid, rows, nrows):
    nu = (nrows // rows) // NS
    for u in range(nu):
        pltpu.sync_copy(zeros_hbm, acc.at[pl.ds((sid * nu + u) * rows, rows)])

    @pl.when(sid == 0)
    def _():
        pltpu.sync_copy(zeros_hbm.at[pl.ds(0, TAIL)],
                        acc.at[pl.ds((nrows // rows) * rows, TAIL)])


def _read_out(acc, stage, out_hbm, cid, sid, rows, nrows):
    nu = (nrows // rows) // NS
    for u in range(nu):
        base = (sid * nu + u) * rows
        pltpu.sync_copy(acc.at[pl.ds(base, rows)], stage)
        pltpu.sync_copy(stage, out_hbm.at[cid, pl.ds(base, rows)])

    @pl.when(sid == 0)
    def _():
        base = (nrows // rows) * rows
        pltpu.sync_copy(acc.at[pl.ds(base, TAIL)], stage.at[pl.ds(0, TAIL)])
        pltpu.sync_copy(stage.at[pl.ds(0, TAIL)], out_hbm.at[cid, pl.ds(base, TAIL)])


def _deg_body(dst_hbm, zeros_hbm, ones_hbm, out_hbm, dst_v, obuf, tbuf, acc):
    cid = lax.axis_index("c")
    sid = lax.axis_index("s")
    wid = cid * NS + sid
    pltpu.sync_copy(ones_hbm, obuf)
    _zero_acc(zeros_hbm, acc, sid, ZR, N)
    plsc.subcore_barrier()

    def chunk(k):
        pltpu.sync_copy(dst_hbm.at[k], dst_v)
        for b in range(NB16):
            pltpu.sync_copy(obuf, acc.at[dst_v.at[b]], add=True)

    full = NCH16 // NW

    def body(j, carry):
        chunk(wid * full + j)
        return carry

    lax.fori_loop(0, full, body, 0)

    @pl.when(wid < NCH16 - NW * full)
    def _():
        chunk(NW * full + wid)

    plsc.subcore_barrier()
    _read_out(acc, tbuf, out_hbm, cid, sid, ZR, N)


_deg_call = pl.kernel(
    _deg_body,
    out_type=jax.ShapeDtypeStruct((NC, N, 1), jnp.float32),
    mesh=_MESH,
    scratch_types=[
        pltpu.VMEM((NB16, IB), jnp.int32),
        pltpu.VMEM((IB, 1), jnp.float32),
        pltpu.VMEM((ZR, 1), jnp.float32),
        pltpu.VMEM_SHARED((N, 1), jnp.float32),
    ],
    compiler_params=_SC_PARAMS,
)


def _mp_body(F, nb, nchunk, src_hbm, dst_hbm, g_hbm, zeros_hbm, out_hbm,
             src_v, dst_v, rows_v, acc, sem):
    cid = lax.axis_index("c")
    sid = lax.axis_index("s")
    wid = cid * NS + sid
    rows = nb * IB
    _zero_acc(zeros_hbm, acc, sid, rows, N)
    plsc.subcore_barrier()

    def chunk(k):
        pltpu.sync_copy(src_hbm.at[k], src_v)
        pltpu.sync_copy(dst_hbm.at[k], dst_v)
        cps = [
            pltpu.async_copy(g_hbm.at[src_v.at[b]], rows_v.at[pl.ds(b * IB, IB)], sem)
            for b in range(nb)
        ]
        for c in cps:
            c.wait()
        for b in range(nb):
            pltpu.sync_copy(rows_v.at[pl.ds(b * IB, IB)], acc.at[dst_v.at[b]], add=True)

    full = nchunk // NW

    def body(j, carry):
        chunk(wid * full + j)
        return carry

    lax.fori_loop(0, full, body, 0)

    @pl.when(wid < nchunk - NW * full)
    def _():
        chunk(NW * full + wid)

    plsc.subcore_barrier()
    _read_out(acc, rows_v, out_hbm, cid, sid, rows, N)


def _make_mp(F, nb, nchunk):
    return pl.kernel(
        functools.partial(_mp_body, F, nb, nchunk),
        out_type=jax.ShapeDtypeStruct((NC, N, F), jnp.float32),
        mesh=_MESH,
        scratch_types=[
            pltpu.VMEM((nb, IB), jnp.int32),
            pltpu.VMEM((nb, IB), jnp.int32),
            pltpu.VMEM((nb * IB, F), jnp.float32),
            pltpu.VMEM_SHARED((N, F), jnp.float32),
            pltpu.SemaphoreType.DMA,
        ],
        compiler_params=_SC_PARAMS,
    )


_mp32 = _make_mp(32, NB32, NCH32)
_mp16 = _make_mp(16, NB16, NCH16)


R = 4104
GRID_R = N // R


def _tc1_body(deg_ref, xf_ref, w1_ref, ws1_ref, dinv_ref, g1_ref, skip_ref):
    deg = deg_ref[0] + deg_ref[1] + 1.0
    dinv = lax.rsqrt(deg)
    xf = xf_ref[...]
    h1 = jnp.dot(xf, w1_ref[...], preferred_element_type=jnp.float32)
    dinv_ref[...] = dinv
    g1_ref[...] = h1 * dinv
    skip_ref[...] = jnp.dot(xf, ws1_ref[...], preferred_element_type=jnp.float32)


def _tc1(deg, xf, W1, Ws1):
    return pl.pallas_call(
        _tc1_body,
        grid=(GRID_R,),
        in_specs=[
            pl.BlockSpec((NC, R, 1), lambda i: (0, i, 0)),
            pl.BlockSpec((R, 64), lambda i: (i, 0)),
            pl.BlockSpec((64, 32), lambda i: (0, 0)),
            pl.BlockSpec((64, 32), lambda i: (0, 0)),
        ],
        out_specs=[
            pl.BlockSpec((R, 1), lambda i: (i, 0)),
            pl.BlockSpec((R, 32), lambda i: (i, 0)),
            pl.BlockSpec((R, 32), lambda i: (i, 0)),
        ],
        out_shape=[
            jax.ShapeDtypeStruct((N, 1), jnp.float32),
            jax.ShapeDtypeStruct((N, 32), jnp.float32),
            jax.ShapeDtypeStruct((N, 32), jnp.float32),
        ],
    )(deg, xf, W1, Ws1)


def _layer_body(acc_ref, g_ref, skip_ref, dinv_ref, b_ref, w_ref, out_ref):
    dinv = dinv_ref[...]
    conv = dinv * (acc_ref[0] + acc_ref[1] + g_ref[...]) + b_ref[...]
    h = jnp.maximum(conv, 0.0) + skip_ref[...]
    hw = jnp.dot(h, w_ref[...], preferred_element_type=jnp.float32)
    out_ref[...] = hw * dinv


def _layer(acc, g, skip, dinv, b, W, fin, fout):
    return pl.pallas_call(
        _layer_body,
        grid=(GRID_R,),
        in_specs=[
            pl.BlockSpec((NC, R, fin), lambda i: (0, i, 0)),
            pl.BlockSpec((R, fin), lambda i: (i, 0)),
            pl.BlockSpec((R, fin), lambda i: (i, 0)),
            pl.BlockSpec((R, 1), lambda i: (i, 0)),
            pl.BlockSpec((fin,), lambda i: (0,)),
            pl.BlockSpec((fin, fout), lambda i: (0, 0)),
        ],
        out_specs=pl.BlockSpec((R, fout), lambda i: (i, 0)),
        out_shape=jax.ShapeDtypeStruct((N, fout), jnp.float32),
    )(acc, g, skip, dinv, b, W)


def _tc3_body(acc_ref, g_ref, dinv_ref, b_ref, w_ref, ws_ref, g3_ref, skip2_ref):
    dinv = dinv_ref[...]
    conv = dinv * (acc_ref[0] + acc_ref[1] + g_ref[...]) + b_ref[...]
    h2 = jnp.maximum(conv, 0.0)
    g3_ref[...] = jnp.dot(h2, w_ref[...], preferred_element_type=jnp.float32) * dinv
    skip2_ref[...] = jnp.dot(h2, ws_ref[...], preferred_element_type=jnp.float32)


def _tc3(acc, g, dinv, b, W, Ws):
    return pl.pallas_call(
        _tc3_body,
        grid=(GRID_R,),
        in_specs=[
            pl.BlockSpec((NC, R, 16), lambda i: (0, i, 0)),
            pl.BlockSpec((R, 16), lambda i: (i, 0)),
            pl.BlockSpec((R, 1), lambda i: (i, 0)),
            pl.BlockSpec((16,), lambda i: (0,)),
            pl.BlockSpec((16, 16), lambda i: (0, 0)),
            pl.BlockSpec((16, 16), lambda i: (0, 0)),
        ],
        out_specs=[
            pl.BlockSpec((R, 16), lambda i: (i, 0)),
            pl.BlockSpec((R, 16), lambda i: (i, 0)),
        ],
        out_shape=[
            jax.ShapeDtypeStruct((N, 16), jnp.float32),
            jax.ShapeDtypeStruct((N, 16), jnp.float32),
        ],
    )(acc, g, dinv, b, W, Ws)


def _tc5_body(acc_ref, g_ref, dinv_ref, b_ref, out_ref):
    conv = dinv_ref[...] * (acc_ref[0] + acc_ref[1] + g_ref[...]) + b_ref[...]
    out_ref[...] = jnp.maximum(conv, 0.0)


def _tc5(acc, g, dinv, b):
    return pl.pallas_call(
        _tc5_body,
        grid=(GRID_R,),
        in_specs=[
            pl.BlockSpec((NC, R, 16), lambda i: (0, i, 0)),
            pl.BlockSpec((R, 16), lambda i: (i, 0)),
            pl.BlockSpec((R, 1), lambda i: (i, 0)),
            pl.BlockSpec((16,), lambda i: (0,)),
        ],
        out_specs=pl.BlockSpec((R, 16), lambda i: (i, 0)),
        out_shape=jax.ShapeDtypeStruct((N, 16), jnp.float32),
    )(acc, g, dinv, b)


FC1_BLK = 256


def _fc1_body(x_ref, w_ref, b_ref, out_ref):
    out_ref[...] = (
        jnp.dot(x_ref[...], w_ref[...], preferred_element_type=jnp.float32)
        + b_ref[...]
    )


def _fc1(x4r, Wf1, bf1):
    return pl.pallas_call(
        _fc1_body,
        grid=(4096 // FC1_BLK,),
        in_specs=[
            pl.BlockSpec((32, 1026 * 16), lambda j: (0, 0)),
            pl.BlockSpec((1026 * 16, FC1_BLK), lambda j: (0, j)),
            pl.BlockSpec((FC1_BLK,), lambda j: (j,)),
        ],
        out_specs=pl.BlockSpec((32, FC1_BLK), lambda j: (0, j)),
        out_shape=jax.ShapeDtypeStruct((32, 4096), jnp.float32),
    )(x4r, Wf1, bf1)


def _fc2_body(x_ref, w_ref, b_ref, out_ref):
    out_ref[...] = (
        jnp.dot(x_ref[...], w_ref[...], preferred_element_type=jnp.float32)
        + b_ref[...]
    )


def _fc2(xfc, Wf2, bf2):
    return pl.pallas_call(
        _fc2_body,
        in_specs=[
            pl.BlockSpec((32, 4096), lambda: (0, 0)),
            pl.BlockSpec((4096, 1026), lambda: (0, 0)),
            pl.BlockSpec((1026,), lambda: (0,)),
        ],
        out_specs=pl.BlockSpec((32, 1026), lambda: (0, 0)),
        out_shape=jax.ShapeDtypeStruct((32, 1026), jnp.float32),
    )(xfc, Wf2, bf2)


def kernel(x, edge_index, W1, b1, Ws1, W2, b2, W3, b3, Ws2, W4, b4,
           Wf1, bf1, Wf2, bf2):
    xf = x.reshape(N, 64)
    ei = edge_index.astype(jnp.int32)
    src16 = ei[0].reshape(NCH16, NB16, IB)
    dst16 = ei[1].reshape(NCH16, NB16, IB)
    src32 = ei[0].reshape(NCH32, NB32, IB)
    dst32 = ei[1].reshape(NCH32, NB32, IB)

    zeros1 = jnp.zeros((ZR, 1), jnp.float32)
    ones1 = jnp.ones((IB, 1), jnp.float32)
    zeros32 = jnp.zeros((NB32 * IB, 32), jnp.float32)
    zeros16 = jnp.zeros((NB16 * IB, 16), jnp.float32)

    deg = _deg_call(dst16, zeros1, ones1)
    dinv, g1, skip = _tc1(deg, xf, W1, Ws1)

    acc1 = _mp32(src32, dst32, g1, zeros32)
    g2 = _layer(acc1, g1, skip, dinv, b1, W2, 32, 16)

    acc2 = _mp16(src16, dst16, g2, zeros16)
    g3, skip2 = _tc3(acc2, g2, dinv, b2, W3, Ws2)

    acc3 = _mp16(src16, dst16, g3, zeros16)
    g4 = _layer(acc3, g3, skip2, dinv, b3, W4, 16, 16)

    acc4 = _mp16(src16, dst16, g4, zeros16)
    x4 = _tc5(acc4, g4, dinv, b4)

    x4r = x4.reshape(32, 1026 * 16)
    xfc = _fc1(x4r, Wf1, bf1)
    pred = _fc2(xfc, Wf2, bf2)
    return (pred, x4r)


# speedup vs baseline: 41.4612x; 1.2500x over previous
import functools

import jax
import jax.numpy as jnp
from jax import lax
from jax.experimental import pallas as pl
from jax.experimental.pallas import tpu as pltpu
from jax.experimental.pallas import tpu_sc as plsc

N = 32832
E = N * 32
IB = 128
NC, NS = 2, 16
NW = NC * NS
NB16 = 16
NB32 = 4
NCH16 = E // (NB16 * IB)
NCH32 = E // (NB32 * IB)
ZR = 2048
TAIL = N - (N // ZR) * ZR

_MESH = plsc.VectorSubcoreMesh(core_axis_name="c", subcore_axis_name="s")
_SC_PARAMS = pltpu.CompilerParams(use_tc_tiling_on_sc=False)


def _zero_acc(zeros_hbm, acc, sid, rows, nrows):
    nu = (nrows // rows) // NS
    for u in range(nu):
        pltpu.sync_copy(zeros_hbm, acc.at[pl.ds((sid * nu + u) * rows, rows)])

    @pl.when(sid == 0)
    def _():
        pltpu.sync_copy(zeros_hbm.at[pl.ds(0, TAIL)],
                        acc.at[pl.ds((nrows // rows) * rows, TAIL)])


def _read_out(acc, stage, out_hbm, cid, sid, rows, nrows):
    nu = (nrows // rows) // NS
    for u in range(nu):
        base = (sid * nu + u) * rows
        pltpu.sync_copy(acc.at[pl.ds(base, rows)], stage)
        pltpu.sync_copy(stage, out_hbm.at[cid, pl.ds(base, rows)])

    @pl.when(sid == 0)
    def _():
        base = (nrows // rows) * rows
        pltpu.sync_copy(acc.at[pl.ds(base, TAIL)], stage.at[pl.ds(0, TAIL)])
        pltpu.sync_copy(stage.at[pl.ds(0, TAIL)], out_hbm.at[cid, pl.ds(base, TAIL)])


def _deg_body(dst_hbm, zeros_hbm, ones_hbm, out_hbm, dst_a, dst_b, obuf, tbuf,
              acc, sem_a, sem_b):
    cid = lax.axis_index("c")
    sid = lax.axis_index("s")
    wid = cid * NS + sid
    pltpu.sync_copy(ones_hbm, obuf)
    _zero_acc(zeros_hbm, acc, sid, ZR, N)
    plsc.subcore_barrier()

    full = NCH16 // NW
    rem = NCH16 - NW * full
    bufs = [(dst_a, sem_a), (dst_b, sem_b)]
    pend = [None, None]
    for t in range(full):
        bi = t % 2
        dv, sm = bufs[bi]
        if pend[bi] is not None:
            for d in pend[bi]:
                d.wait()
        pltpu.sync_copy(dst_hbm.at[wid * full + t], dv)
        pend[bi] = [
            pltpu.async_copy(obuf, acc.at[dv.at[b]], sm, add=True)
            for b in range(NB16)
        ]
    for bi in (0, 1):
        if pend[bi] is not None:
            for d in pend[bi]:
                d.wait()

    @pl.when(wid < rem)
    def _():
        dv, sm = bufs[0]
        pltpu.sync_copy(dst_hbm.at[NW * full + wid], dv)
        tail = [
            pltpu.async_copy(obuf, acc.at[dv.at[b]], sm, add=True)
            for b in range(NB16)
        ]
        for d in tail:
            d.wait()

    plsc.subcore_barrier()
    _read_out(acc, tbuf, out_hbm, cid, sid, ZR, N)


_deg_call = pl.kernel(
    _deg_body,
    out_type=jax.ShapeDtypeStruct((NC, N, 1), jnp.float32),
    mesh=_MESH,
    scratch_types=[
        pltpu.VMEM((NB16, IB), jnp.int32),
        pltpu.VMEM((NB16, IB), jnp.int32),
        pltpu.VMEM((IB, 1), jnp.float32),
        pltpu.VMEM((ZR, 1), jnp.float32),
        pltpu.VMEM_SHARED((N, 1), jnp.float32),
        pltpu.SemaphoreType.DMA,
        pltpu.SemaphoreType.DMA,
    ],
    compiler_params=_SC_PARAMS,
)


def _mp_body(F, nb, nchunk, src_hbm, dst_hbm, g_hbm, zeros_hbm, out_hbm,
             src_a, dst_a, rows_a, src_b, dst_b, rows_b, acc,
             gsem_a, gsem_b, ssem):
    cid = lax.axis_index("c")
    sid = lax.axis_index("s")
    wid = cid * NS + sid
    rows = nb * IB
    full = nchunk // NW
    rem = nchunk - NW * full
    base = wid * full
    bufs = [(src_a, dst_a, rows_a, gsem_a), (src_b, dst_b, rows_b, gsem_b)]
    pend_g = [None, None]
    pend_s = [None, None]

    def fire_chunk(k, bi):
        sv, dv, rv, gs = bufs[bi]
        pltpu.sync_copy(src_hbm.at[k], sv)
        pltpu.sync_copy(dst_hbm.at[k], dv)
        pend_g[bi] = [
            pltpu.async_copy(g_hbm.at[sv.at[b]], rv.at[pl.ds(b * IB, IB)], gs)
            for b in range(nb)
        ]

    def scatter_chunk(bi):
        sv, dv, rv, gs = bufs[bi]
        for d in pend_g[bi]:
            d.wait()
        pend_g[bi] = None
        pend_s[bi] = [
            pltpu.async_copy(rv.at[pl.ds(b * IB, IB)], acc.at[dv.at[b]], ssem,
                             add=True)
            for b in range(nb)
        ]

    def drain_s(bi):
        for d in pend_s[bi]:
            d.wait()
        pend_s[bi] = None

    fire_chunk(base, 0)
    _zero_acc(zeros_hbm, acc, sid, rows, N)
    plsc.subcore_barrier()

    for t in range(1, full):
        bi = t % 2
        if pend_s[bi] is not None:
            drain_s(bi)
        fire_chunk(base + t, bi)
        scatter_chunk(1 - bi)
    scatter_chunk((full - 1) % 2)
    for bi in (0, 1):
        if pend_s[bi] is not None:
            drain_s(bi)

    @pl.when(wid < rem)
    def _():
        sv, dv, rv, gs = bufs[0]
        k = NW * full + wid
        pltpu.sync_copy(src_hbm.at[k], sv)
        pltpu.sync_copy(dst_hbm.at[k], dv)
        tg = [
            pltpu.async_copy(g_hbm.at[sv.at[b]], rv.at[pl.ds(b * IB, IB)], gs)
            for b in range(nb)
        ]
        for d in tg:
            d.wait()
        ts = [
            pltpu.async_copy(rv.at[pl.ds(b * IB, IB)], acc.at[dv.at[b]], ssem,
                             add=True)
            for b in range(nb)
        ]
        for d in ts:
            d.wait()

    plsc.subcore_barrier()
    _read_out(acc, rows_a, out_hbm, cid, sid, rows, N)


def _make_mp(F, nb, nchunk):
    return pl.kernel(
        functools.partial(_mp_body, F, nb, nchunk),
        out_type=jax.ShapeDtypeStruct((NC, N, F), jnp.float32),
        mesh=_MESH,
        scratch_types=[
            pltpu.VMEM((nb, IB), jnp.int32),
            pltpu.VMEM((nb, IB), jnp.int32),
            pltpu.VMEM((nb * IB, F), jnp.float32),
            pltpu.VMEM((nb, IB), jnp.int32),
            pltpu.VMEM((nb, IB), jnp.int32),
            pltpu.VMEM((nb * IB, F), jnp.float32),
            pltpu.VMEM_SHARED((N, F), jnp.float32),
            pltpu.SemaphoreType.DMA,
            pltpu.SemaphoreType.DMA,
            pltpu.SemaphoreType.DMA,
        ],
        compiler_params=_SC_PARAMS,
    )


_mp32 = _make_mp(32, NB32, NCH32)
_mp16 = _make_mp(16, NB16, NCH16)


R = 4104
GRID_R = N // R


def _tc1_body(deg_ref, xf_ref, w1_ref, ws1_ref, dinv_ref, g1_ref, skip_ref):
    deg = deg_ref[0] + deg_ref[1] + 1.0
    dinv = lax.rsqrt(deg)
    xf = xf_ref[...]
    h1 = jnp.dot(xf, w1_ref[...], preferred_element_type=jnp.float32)
    dinv_ref[...] = dinv
    g1_ref[...] = h1 * dinv
    skip_ref[...] = jnp.dot(xf, ws1_ref[...], preferred_element_type=jnp.float32)


def _tc1(deg, xf, W1, Ws1):
    return pl.pallas_call(
        _tc1_body,
        grid=(GRID_R,),
        in_specs=[
            pl.BlockSpec((NC, R, 1), lambda i: (0, i, 0)),
            pl.BlockSpec((R, 64), lambda i: (i, 0)),
            pl.BlockSpec((64, 32), lambda i: (0, 0)),
            pl.BlockSpec((64, 32), lambda i: (0, 0)),
        ],
        out_specs=[
            pl.BlockSpec((R, 1), lambda i: (i, 0)),
            pl.BlockSpec((R, 32), lambda i: (i, 0)),
            pl.BlockSpec((R, 32), lambda i: (i, 0)),
        ],
        out_shape=[
            jax.ShapeDtypeStruct((N, 1), jnp.float32),
            jax.ShapeDtypeStruct((N, 32), jnp.float32),
            jax.ShapeDtypeStruct((N, 32), jnp.float32),
        ],
    )(deg, xf, W1, Ws1)


def _layer_body(acc_ref, g_ref, skip_ref, dinv_ref, b_ref, w_ref, out_ref):
    dinv = dinv_ref[...]
    conv = dinv * (acc_ref[0] + acc_ref[1] + g_ref[...]) + b_ref[...]
    h = jnp.maximum(conv, 0.0) + skip_ref[...]
    hw = jnp.dot(h, w_ref[...], preferred_element_type=jnp.float32)
    out_ref[...] = hw * dinv


def _layer(acc, g, skip, dinv, b, W, fin, fout):
    return pl.pallas_call(
        _layer_body,
        grid=(GRID_R,),
        in_specs=[
            pl.BlockSpec((NC, R, fin), lambda i: (0, i, 0)),
            pl.BlockSpec((R, fin), lambda i: (i, 0)),
            pl.BlockSpec((R, fin), lambda i: (i, 0)),
            pl.BlockSpec((R, 1), lambda i: (i, 0)),
            pl.BlockSpec((fin,), lambda i: (0,)),
            pl.BlockSpec((fin, fout), lambda i: (0, 0)),
        ],
        out_specs=pl.BlockSpec((R, fout), lambda i: (i, 0)),
        out_shape=jax.ShapeDtypeStruct((N, fout), jnp.float32),
    )(acc, g, skip, dinv, b, W)


def _tc3_body(acc_ref, g_ref, dinv_ref, b_ref, w_ref, ws_ref, g3_ref, skip2_ref):
    dinv = dinv_ref[...]
    conv = dinv * (acc_ref[0] + acc_ref[1] + g_ref[...]) + b_ref[...]
    h2 = jnp.maximum(conv, 0.0)
    g3_ref[...] = jnp.dot(h2, w_ref[...], preferred_element_type=jnp.float32) * dinv
    skip2_ref[...] = jnp.dot(h2, ws_ref[...], preferred_element_type=jnp.float32)


def _tc3(acc, g, dinv, b, W, Ws):
    return pl.pallas_call(
        _tc3_body,
        grid=(GRID_R,),
        in_specs=[
            pl.BlockSpec((NC, R, 16), lambda i: (0, i, 0)),
            pl.BlockSpec((R, 16), lambda i: (i, 0)),
            pl.BlockSpec((R, 1), lambda i: (i, 0)),
            pl.BlockSpec((16,), lambda i: (0,)),
            pl.BlockSpec((16, 16), lambda i: (0, 0)),
            pl.BlockSpec((16, 16), lambda i: (0, 0)),
        ],
        out_specs=[
            pl.BlockSpec((R, 16), lambda i: (i, 0)),
            pl.BlockSpec((R, 16), lambda i: (i, 0)),
        ],
        out_shape=[
            jax.ShapeDtypeStruct((N, 16), jnp.float32),
            jax.ShapeDtypeStruct((N, 16), jnp.float32),
        ],
    )(acc, g, dinv, b, W, Ws)


def _tc5_body(acc_ref, g_ref, dinv_ref, b_ref, out_ref):
    conv = dinv_ref[...] * (acc_ref[0] + acc_ref[1] + g_ref[...]) + b_ref[...]
    out_ref[...] = jnp.maximum(conv, 0.0)


def _tc5(acc, g, dinv, b):
    return pl.pallas_call(
        _tc5_body,
        grid=(GRID_R,),
        in_specs=[
            pl.BlockSpec((NC, R, 16), lambda i: (0, i, 0)),
            pl.BlockSpec((R, 16), lambda i: (i, 0)),
            pl.BlockSpec((R, 1), lambda i: (i, 0)),
            pl.BlockSpec((16,), lambda i: (0,)),
        ],
        out_specs=pl.BlockSpec((R, 16), lambda i: (i, 0)),
        out_shape=jax.ShapeDtypeStruct((N, 16), jnp.float32),
    )(acc, g, dinv, b)


FC1_BLK = 256


def _fc1_body(x_ref, w_ref, b_ref, out_ref):
    out_ref[...] = (
        jnp.dot(x_ref[...], w_ref[...], preferred_element_type=jnp.float32)
        + b_ref[...]
    )


def _fc1(x4r, Wf1, bf1):
    return pl.pallas_call(
        _fc1_body,
        grid=(4096 // FC1_BLK,),
        in_specs=[
            pl.BlockSpec((32, 1026 * 16), lambda j: (0, 0)),
            pl.BlockSpec((1026 * 16, FC1_BLK), lambda j: (0, j)),
            pl.BlockSpec((FC1_BLK,), lambda j: (j,)),
        ],
        out_specs=pl.BlockSpec((32, FC1_BLK), lambda j: (0, j)),
        out_shape=jax.ShapeDtypeStruct((32, 4096), jnp.float32),
    )(x4r, Wf1, bf1)


def _fc2_body(x_ref, w_ref, b_ref, out_ref):
    out_ref[...] = (
        jnp.dot(x_ref[...], w_ref[...], preferred_element_type=jnp.float32)
        + b_ref[...]
    )


def _fc2(xfc, Wf2, bf2):
    return pl.pallas_call(
        _fc2_body,
        in_specs=[
            pl.BlockSpec((32, 4096), lambda: (0, 0)),
            pl.BlockSpec((4096, 1026), lambda: (0, 0)),
            pl.BlockSpec((1026,), lambda: (0,)),
        ],
        out_specs=pl.BlockSpec((32, 1026), lambda: (0, 0)),
        out_shape=jax.ShapeDtypeStruct((32, 1026), jnp.float32),
    )(xfc, Wf2, bf2)


def kernel(x, edge_index, W1, b1, Ws1, W2, b2, W3, b3, Ws2, W4, b4,
           Wf1, bf1, Wf2, bf2):
    xf = x.reshape(N, 64)
    ei = edge_index.astype(jnp.int32)
    src16 = ei[0].reshape(NCH16, NB16, IB)
    dst16 = ei[1].reshape(NCH16, NB16, IB)
    src32 = ei[0].reshape(NCH32, NB32, IB)
    dst32 = ei[1].reshape(NCH32, NB32, IB)

    zeros1 = jnp.zeros((ZR, 1), jnp.float32)
    ones1 = jnp.ones((IB, 1), jnp.float32)
    zeros32 = jnp.zeros((NB32 * IB, 32), jnp.float32)
    zeros16 = jnp.zeros((NB16 * IB, 16), jnp.float32)

    deg = _deg_call(dst16, zeros1, ones1)
    dinv, g1, skip = _tc1(deg, xf, W1, Ws1)

    acc1 = _mp32(src32, dst32, g1, zeros32)
    g2 = _layer(acc1, g1, skip, dinv, b1, W2, 32, 16)

    acc2 = _mp16(src16, dst16, g2, zeros16)
    g3, skip2 = _tc3(acc2, g2, dinv, b2, W3, Ws2)

    acc3 = _mp16(src16, dst16, g3, zeros16)
    g4 = _layer(acc3, g3, skip2, dinv, b3, W4, 16, 16)

    acc4 = _mp16(src16, dst16, g4, zeros16)
    x4 = _tc5(acc4, g4, dinv, b4)

    x4r = x4.reshape(32, 1026 * 16)
    xfc = _fc1(x4r, Wf1, bf1)
    pred = _fc2(xfc, Wf2, bf2)
    return (pred, x4r)
